# Initial kernel scaffold; baseline (speedup 1.0000x reference)
import functools

import jax
import jax.numpy as jnp
from jax import lax
from jax.experimental import pallas as pl
from jax.experimental.pallas import tpu as pltpu
from jax.experimental.pallas import tpu_sc as plsc

N_NODES = 10000
N_PAD = 10240
TRASH = 10000
E_EDGES = 320000
K = 128
E_PAD = 323584
NC, NS = 2, 16
RB = 1024
GRID = N_PAD // RB

_MESH = lambda: plsc.VectorSubcoreMesh(core_axis_name="c", subcore_axis_name="s")


def _sc_degree(dst_pad, zeros_tab):
    per_w = E_PAD // (NC * NS)
    nblk = per_w // K

    @functools.partial(
        pl.kernel,
        out_type=jax.ShapeDtypeStruct((NC * NS, N_PAD), jnp.float32),
        mesh=_MESH(),
        scratch_types=[
            pltpu.VMEM((N_PAD,), jnp.float32),
            pltpu.VMEM((K,), jnp.int32),
        ],
    )
    def k(dst_hbm, ztab_hbm, out_hbm, tab_v, idx_v):
        cid = lax.axis_index("c")
        sid = lax.axis_index("s")
        wid = sid * NC + cid
        base = wid * per_w
        pltpu.sync_copy(ztab_hbm, tab_v)
        ones = jnp.full((16,), 1.0, jnp.float32)

        @pl.loop(0, nblk)
        def _(i):
            pltpu.sync_copy(dst_hbm.at[pl.ds(base + i * K, K)], idx_v)
            for j in range(K // 16):
                idx = idx_v[pl.ds(j * 16, 16)]
                plsc.addupdate_scatter(tab_v, [idx], ones)

        pltpu.sync_copy(tab_v, out_hbm.at[wid])

    return k(dst_pad, zeros_tab)


def _sc_aggregate(table, src_pad, dst_pad, zeros_blk, channel_split):
    C = table.shape[-1]
    rows_sub = N_PAD // NS
    if channel_split:
        per_w = E_PAD // NS
    else:
        per_w = E_PAD // (NC * NS)
    nblk = per_w // K

    @functools.partial(
        pl.kernel,
        out_type=jax.ShapeDtypeStruct((NC, N_PAD, C), jnp.float32),
        mesh=_MESH(),
        scratch_types=[
            pltpu.VMEM((K,), jnp.int32),
            pltpu.VMEM((K,), jnp.int32),
            pltpu.VMEM((K, C), jnp.float32),
            pltpu.VMEM_SHARED((N_PAD, C), jnp.float32),
            pltpu.SemaphoreType.DMA,
        ],
    )
    def k(h_hbm, src_hbm, dst_hbm, z_hbm, out_hbm, sidx_v, didx_v, rows_v,
          acc_sh, sem):
        cid = lax.axis_index("c")
        sid = lax.axis_index("s")
        pltpu.sync_copy(z_hbm, acc_sh.at[pl.ds(sid * rows_sub, rows_sub)])
        plsc.subcore_barrier()

        def edge_loop(tab2d, base):
            @pl.loop(0, nblk)
            def _(i):
                off = base + i * K
                pltpu.sync_copy(src_hbm.at[pl.ds(off, K)], sidx_v)
                pltpu.sync_copy(dst_hbm.at[pl.ds(off, K)], didx_v)
                pltpu.async_copy(tab2d.at[sidx_v], rows_v, sem).wait()
                pltpu.sync_copy(rows_v, acc_sh.at[didx_v], add=True)

        if channel_split:
            base = sid * per_w

            @pl.when(cid == 0)
            def _():
                edge_loop(h_hbm.at[0], base)

            @pl.when(cid == 1)
            def _():
                edge_loop(h_hbm.at[1], base)
        else:
            wid = sid * NC + cid
            edge_loop(h_hbm, wid * per_w)

        plsc.subcore_barrier()
        sl = pl.ds(sid * rows_sub, rows_sub)
        pltpu.sync_copy(acc_sh.at[sl], out_hbm.at[cid].at[sl])

    return k(table, src_pad, dst_pad, zeros_blk)


def _tc_call(body, out_shapes, *args):
    in_specs = []
    for a in args:
        if a.ndim == 1:
            in_specs.append(pl.BlockSpec(a.shape, lambda i: (0,)))
        elif a.shape[0] == N_PAD:
            bs = (RB,) + a.shape[1:]
            nd = a.ndim
            in_specs.append(pl.BlockSpec(bs, lambda i, _n=nd: (i,) + (0,) * (_n - 1)))
        elif a.ndim == 3:
            in_specs.append(pl.BlockSpec((a.shape[0], RB, a.shape[2]),
                                         lambda i: (0, i, 0)))
        elif a.shape[-1] == N_PAD:
            in_specs.append(pl.BlockSpec((a.shape[0], RB), lambda i: (0, i)))
        else:
            nd = a.ndim
            in_specs.append(pl.BlockSpec(a.shape, lambda i, _n=nd: (0,) * _n))
    out_specs = []
    for s in out_shapes:
        if len(s.shape) == 3:
            out_specs.append(pl.BlockSpec((s.shape[0], RB, s.shape[2]),
                                          lambda i: (0, i, 0)))
        else:
            nd = len(s.shape)
            out_specs.append(pl.BlockSpec((RB,) + s.shape[1:],
                                          lambda i, _n=nd: (i,) + (0,) * (_n - 1)))
    return pl.pallas_call(
        body,
        grid=(GRID,),
        in_specs=in_specs,
        out_specs=out_specs,
        out_shape=list(out_shapes),
    )(*args)


def _tc0_body(parts_ref, x_ref, dinv_ref, dinv2_ref, xs_ref):
    deg = jnp.sum(parts_ref[...], axis=0) + 1.0
    di = lax.rsqrt(deg)[:, None]
    dinv_ref[...] = di
    dinv2_ref[...] = di * di
    xs_ref[...] = x_ref[...] * di


def _tc1_body(p_ref, x_ref, dinv_ref, dinv2_ref, w1_ref, b1_ref,
              h1_ref, h1s_ref):
    di = dinv_ref[...]
    g1 = (p_ref[0] + p_ref[1]) * di + x_ref[...] * dinv2_ref[...]
    a = jnp.dot(g1, w1_ref[...], preferred_element_type=jnp.float32) + b1_ref[...]
    h1 = jnp.maximum(a, 0.0)
    h1_ref[...] = h1
    hs = h1 * di
    h1s_ref[...] = jnp.stack([hs[:, :128], hs[:, 128:]], axis=0)


def _tc2_body(q_ref, h1_ref, dinv_ref, dinv2_ref, w2_ref, b2_ref, w3_ref,
              u_ref, us_ref):
    di = dinv_ref[...]
    g2 = (jnp.concatenate([q_ref[0], q_ref[1]], axis=1) * di
          + h1_ref[...] * dinv2_ref[...])
    a = jnp.dot(g2, w2_ref[...], preferred_element_type=jnp.float32) + b2_ref[...]
    h2 = jnp.maximum(a, 0.0)
    u = jnp.dot(h2, w3_ref[...], preferred_element_type=jnp.float32)
    u_ref[...] = u
    us_ref[...] = u * di


def _tc3_body(r_ref, u_ref, dinv_ref, dinv2_ref, b3_ref, o_ref):
    g3 = ((r_ref[0] + r_ref[1]) * dinv_ref[...]
          + u_ref[...] * dinv2_ref[...] + b3_ref[...])
    o_ref[...] = jax.nn.sigmoid(g3)


def kernel(x, edge_index, W1, b1, W2, b2, W3, b3):
    f32 = jnp.float32
    src = edge_index[0].astype(jnp.int32)
    dst = edge_index[1].astype(jnp.int32)
    npad = E_PAD - E_EDGES
    src_p = jnp.concatenate([src, jnp.zeros((npad,), jnp.int32)])
    dst_p = jnp.concatenate([dst, jnp.full((npad,), TRASH, jnp.int32)])
    x_p = jnp.pad(x, ((0, N_PAD - N_NODES), (0, 0)))

    zeros_tab = jnp.zeros((N_PAD,), f32)
    zeros128 = jnp.zeros((N_PAD // NS, 128), f32)
    zeros64 = jnp.zeros((N_PAD // NS, 64), f32)

    deg_parts = _sc_degree(dst_p, zeros_tab)

    sds = jax.ShapeDtypeStruct
    dinv, dinv2, xs = _tc_call(
        _tc0_body,
        [sds((N_PAD, 1), f32), sds((N_PAD, 1), f32), sds((N_PAD, 128), f32)],
        deg_parts, x_p)

    p1 = _sc_aggregate(xs, src_p, dst_p, zeros128, channel_split=False)

    h1, h1s = _tc_call(
        _tc1_body,
        [sds((N_PAD, 256), f32), sds((2, N_PAD, 128), f32)],
        p1, x_p, dinv, dinv2, W1, b1)

    q2 = _sc_aggregate(h1s, src_p, dst_p, zeros128, channel_split=True)

    u, us = _tc_call(
        _tc2_body,
        [sds((N_PAD, 64), f32), sds((N_PAD, 64), f32)],
        q2, h1, dinv, dinv2, W2, b2, W3)

    r3 = _sc_aggregate(us, src_p, dst_p, zeros64, channel_split=False)

    out = _tc_call(
        _tc3_body,
        [sds((N_PAD, 64), f32)],
        r3, u, dinv, dinv2, b3)[0]

    return out[:N_NODES]


# speedup vs baseline: 10.0825x; 10.0825x over previous
import dataclasses
import functools

import jax
import jax.numpy as jnp
from jax import lax
from jax.experimental import pallas as pl
from jax.experimental.pallas import tpu as pltpu
from jax.experimental.pallas import tpu_sc as plsc

N_NODES = 10000
N_PAD = 10240
TRASH = 10000
E_EDGES = 320000
K = 128
E_PAD = 323584
NC, NS = 2, 16
RB = 1024
GRID = N_PAD // RB

_MESH = lambda: plsc.VectorSubcoreMesh(core_axis_name="c", subcore_axis_name="s")


def _sc_params():
    cp = pltpu.CompilerParams()
    fields = pltpu.CompilerParams.__dataclass_fields__
    if "needs_layout_passes" in fields:
        cp = dataclasses.replace(cp, needs_layout_passes=False)
    if "use_tc_tiling_on_sc" in fields:
        cp = dataclasses.replace(cp, use_tc_tiling_on_sc=False)
    return cp


def _sc_degree(dst_pad, zeros_tab):
    per_w = E_PAD // (NC * NS)
    nblk = per_w // K

    @functools.partial(
        pl.kernel,
        out_type=jax.ShapeDtypeStruct((NC * NS, N_PAD), jnp.float32),
        mesh=_MESH(),
        compiler_params=_sc_params(),
        scratch_types=[
            pltpu.VMEM((N_PAD,), jnp.float32),
            pltpu.VMEM((K,), jnp.int32),
        ],
    )
    def k(dst_hbm, ztab_hbm, out_hbm, tab_v, idx_v):
        cid = lax.axis_index("c")
        sid = lax.axis_index("s")
        wid = sid * NC + cid
        base = wid * per_w
        pltpu.sync_copy(ztab_hbm, tab_v)
        ones = jnp.full((16,), 1.0, jnp.float32)

        @pl.loop(0, nblk)
        def _(i):
            pltpu.sync_copy(dst_hbm.at[pl.ds(base + i * K, K)], idx_v)
            for j in range(K // 16):
                idx = idx_v[pl.ds(j * 16, 16)]
                plsc.addupdate_scatter(tab_v, [idx], ones)

        pltpu.sync_copy(tab_v, out_hbm.at[wid])

    return k(dst_pad, zeros_tab)


def _sc_aggregate(table, src_pad, dst_pad, zeros_blk, channel_split):
    C = table.shape[-1]
    rows_sub = N_PAD // NS
    if channel_split:
        per_w = E_PAD // NS
    else:
        per_w = E_PAD // (NC * NS)
    nblk = per_w // K

    @functools.partial(
        pl.kernel,
        out_type=jax.ShapeDtypeStruct((NC, N_PAD, C), jnp.float32),
        mesh=_MESH(),
        compiler_params=_sc_params(),
        scratch_types=[
            pltpu.VMEM((K,), jnp.int32),
            pltpu.VMEM((K,), jnp.int32),
            pltpu.VMEM((K, C), jnp.float32),
            pltpu.VMEM_SHARED((N_PAD, C), jnp.float32),
            pltpu.SemaphoreType.DMA,
        ],
    )
    def k(h_hbm, src_hbm, dst_hbm, z_hbm, out_hbm, sidx_v, didx_v, rows_v,
          acc_sh, sem):
        cid = lax.axis_index("c")
        sid = lax.axis_index("s")
        pltpu.sync_copy(z_hbm, acc_sh.at[pl.ds(sid * rows_sub, rows_sub)])
        plsc.subcore_barrier()

        def edge_loop(tab2d, base):
            @pl.loop(0, nblk)
            def _(i):
                off = base + i * K
                pltpu.sync_copy(src_hbm.at[pl.ds(off, K)], sidx_v)
                pltpu.sync_copy(dst_hbm.at[pl.ds(off, K)], didx_v)
                pltpu.async_copy(tab2d.at[sidx_v], rows_v, sem).wait()
                pltpu.sync_copy(rows_v, acc_sh.at[didx_v], add=True)

        if channel_split:
            base = sid * per_w

            @pl.when(cid == 0)
            def _():
                edge_loop(h_hbm.at[0], base)

            @pl.when(cid == 1)
            def _():
                edge_loop(h_hbm.at[1], base)
        else:
            wid = sid * NC + cid
            edge_loop(h_hbm, wid * per_w)

        plsc.subcore_barrier()
        sl = pl.ds(sid * rows_sub, rows_sub)
        pltpu.sync_copy(acc_sh.at[sl], out_hbm.at[cid].at[sl])

    return k(table, src_pad, dst_pad, zeros_blk)


def _tc_call(body, out_shapes, *args):
    in_specs = []
    for a in args:
        if a.ndim == 1:
            in_specs.append(pl.BlockSpec(a.shape, lambda i: (0,)))
        elif a.shape[0] == N_PAD:
            bs = (RB,) + a.shape[1:]
            nd = a.ndim
            in_specs.append(pl.BlockSpec(bs, lambda i, _n=nd: (i,) + (0,) * (_n - 1)))
        elif a.ndim == 3:
            in_specs.append(pl.BlockSpec((a.shape[0], RB, a.shape[2]),
                                         lambda i: (0, i, 0)))
        elif a.shape[-1] == N_PAD:
            in_specs.append(pl.BlockSpec((a.shape[0], RB), lambda i: (0, i)))
        else:
            nd = a.ndim
            in_specs.append(pl.BlockSpec(a.shape, lambda i, _n=nd: (0,) * _n))
    out_specs = []
    for s in out_shapes:
        if len(s.shape) == 3:
            out_specs.append(pl.BlockSpec((s.shape[0], RB, s.shape[2]),
                                          lambda i: (0, i, 0)))
        else:
            nd = len(s.shape)
            out_specs.append(pl.BlockSpec((RB,) + s.shape[1:],
                                          lambda i, _n=nd: (i,) + (0,) * (_n - 1)))
    return pl.pallas_call(
        body,
        grid=(GRID,),
        in_specs=in_specs,
        out_specs=out_specs,
        out_shape=list(out_shapes),
    )(*args)


def _tc0_body(parts_ref, x_ref, dinv_ref, dinv2_ref, xs_ref):
    deg = jnp.sum(parts_ref[...], axis=0) + 1.0
    di = lax.rsqrt(deg)[:, None]
    dinv_ref[...] = di
    dinv2_ref[...] = di * di
    xs_ref[...] = x_ref[...] * di


def _tc1_body(p_ref, x_ref, dinv_ref, dinv2_ref, w1_ref, b1_ref,
              h1_ref, h1s_ref):
    di = dinv_ref[...]
    g1 = (p_ref[0] + p_ref[1]) * di + x_ref[...] * dinv2_ref[...]
    a = jnp.dot(g1, w1_ref[...], preferred_element_type=jnp.float32) + b1_ref[...]
    h1 = jnp.maximum(a, 0.0)
    h1_ref[...] = h1
    hs = h1 * di
    h1s_ref[...] = jnp.stack([hs[:, :128], hs[:, 128:]], axis=0)


def _tc2_body(q_ref, h1_ref, dinv_ref, dinv2_ref, w2_ref, b2_ref, w3_ref,
              u_ref, us_ref):
    di = dinv_ref[...]
    g2 = (jnp.concatenate([q_ref[0], q_ref[1]], axis=1) * di
          + h1_ref[...] * dinv2_ref[...])
    a = jnp.dot(g2, w2_ref[...], preferred_element_type=jnp.float32) + b2_ref[...]
    h2 = jnp.maximum(a, 0.0)
    u = jnp.dot(h2, w3_ref[...], preferred_element_type=jnp.float32)
    u_ref[...] = u
    us_ref[...] = u * di


def _tc3_body(r_ref, u_ref, dinv_ref, dinv2_ref, b3_ref, o_ref):
    g3 = ((r_ref[0] + r_ref[1]) * dinv_ref[...]
          + u_ref[...] * dinv2_ref[...] + b3_ref[...])
    o_ref[...] = jax.nn.sigmoid(g3)


def kernel(x, edge_index, W1, b1, W2, b2, W3, b3):
    f32 = jnp.float32
    src = edge_index[0].astype(jnp.int32)
    dst = edge_index[1].astype(jnp.int32)
    npad = E_PAD - E_EDGES
    src_p = jnp.concatenate([src, jnp.zeros((npad,), jnp.int32)])
    dst_p = jnp.concatenate([dst, jnp.full((npad,), TRASH, jnp.int32)])
    x_p = jnp.pad(x, ((0, N_PAD - N_NODES), (0, 0)))

    zeros_tab = jnp.zeros((N_PAD,), f32)
    zeros128 = jnp.zeros((N_PAD // NS, 128), f32)
    zeros64 = jnp.zeros((N_PAD // NS, 64), f32)

    deg_parts = _sc_degree(dst_p, zeros_tab)

    sds = jax.ShapeDtypeStruct
    dinv, dinv2, xs = _tc_call(
        _tc0_body,
        [sds((N_PAD, 1), f32), sds((N_PAD, 1), f32), sds((N_PAD, 128), f32)],
        deg_parts, x_p)

    p1 = _sc_aggregate(xs, src_p, dst_p, zeros128, channel_split=False)

    h1, h1s = _tc_call(
        _tc1_body,
        [sds((N_PAD, 256), f32), sds((2, N_PAD, 128), f32)],
        p1, x_p, dinv, dinv2, W1, b1)

    q2 = _sc_aggregate(h1s, src_p, dst_p, zeros128, channel_split=True)

    u, us = _tc_call(
        _tc2_body,
        [sds((N_PAD, 64), f32), sds((N_PAD, 64), f32)],
        q2, h1, dinv, dinv2, W2, b2, W3)

    r3 = _sc_aggregate(us, src_p, dst_p, zeros64, channel_split=False)

    out = _tc_call(
        _tc3_body,
        [sds((N_PAD, 64), f32)],
        r3, u, dinv, dinv2, b3)[0]

    return out[:N_NODES]
